# Initial kernel scaffold; baseline (speedup 1.0000x reference)
import jax
import jax.numpy as jnp
from jax.experimental import pallas as pl


def kernel(x, patch_w, vec32, vec128, wqkv, bqkv, attn_mask, proj_w, fc1_w, fc2_w, head_w):
    raise NotImplementedError("write your pallas kernel here")


# speedup vs baseline: 75.8631x; 75.8631x over previous
import functools
import numpy as np
import jax
import jax.numpy as jnp
from jax.experimental import pallas as pl
from jax.experimental.pallas import tpu as pltpu

_D = 32
_H = 4
_HD = _D // _H
_N = 5
_PATCH = 16
_CHANS = 3
_IMG = 32
_KFLAT = _CHANS * _IMG * _IMG
_HIDDEN = 128
_NUM_CLASSES = 10
_EPS = 1e-6
_GELU_C = float(np.sqrt(2.0 / np.pi))


def _layernorm(v, w, b):
    mu = jnp.mean(v, axis=-1, keepdims=True)
    d = v - mu
    var = jnp.mean(d * d, axis=-1, keepdims=True)
    return d * jax.lax.rsqrt(var + _EPS) * w + b


def _gelu_tanh(v):
    return 0.5 * v * (1.0 + jnp.tanh(_GELU_C * (v + 0.044715 * v * v * v)))


def _vit_kernel(xb_ref, wbig_ref, vec32_ref, vec128_ref, wqkv_ref, bqkv_ref,
                red_ref, projw_ref, fc1w_ref, fc2w_ref, headw_ref, o_ref,
                *, tb, depth):
    scale = float(_HD) ** -0.5

    def vrow(r):
        return vec32_ref[pl.ds(r, 1), :]

    emb = jnp.dot(xb_ref[...], wbig_ref[...],
                  preferred_element_type=jnp.float32)

    toks = [jnp.broadcast_to(vrow(0), (tb, _D))]
    for p in range(_N - 1):
        toks.append(emb[:, p * _D:(p + 1) * _D] + vrow(p + 1))
    x = jnp.concatenate(toks, axis=0)

    for l in range(depth):
        base = _N + 6 * l
        h = _layernorm(x, vrow(base), vrow(base + 1))
        qkv = jnp.dot(h, wqkv_ref[l],
                      preferred_element_type=jnp.float32) + bqkv_ref[l]
        q = qkv[:, 0:_D] * scale
        k = qkv[:, _D:2 * _D]
        v = qkv[:, 2 * _D:3 * _D]

        ctxs = []
        for a in range(_N):
            qa = q[a * tb:(a + 1) * tb]
            logits = [jnp.dot(qa * k[b * tb:(b + 1) * tb], red_ref[...],
                              preferred_element_type=jnp.float32)
                      for b in range(_N)]
            m = logits[0]
            for b in range(1, _N):
                m = jnp.maximum(m, logits[b])
            exps = [jnp.exp(lg - m) for lg in logits]
            z = exps[0]
            for b in range(1, _N):
                z = z + exps[b]
            num = exps[0] * v[0:tb]
            for b in range(1, _N):
                num = num + exps[b] * v[b * tb:(b + 1) * tb]
            ctxs.append(num / z)
        ctx = jnp.concatenate(ctxs, axis=0)
        x = x + jnp.dot(ctx, projw_ref[l],
                        preferred_element_type=jnp.float32) + vrow(base + 2)

        hm = _layernorm(x, vrow(base + 3), vrow(base + 4))
        hm = jnp.dot(hm, fc1w_ref[l],
                     preferred_element_type=jnp.float32) + vec128_ref[pl.ds(l, 1), :]
        hm = _gelu_tanh(hm)
        x = x + jnp.dot(hm, fc2w_ref[l],
                        preferred_element_type=jnp.float32) + vrow(base + 5)

    nrow = _N + 6 * depth
    cls = _layernorm(x[0:tb], vrow(nrow), vrow(nrow + 1))
    o_ref[...] = jnp.dot(cls, headw_ref[...],
                         preferred_element_type=jnp.float32) + vec128_ref[pl.ds(depth, 1), :]


def kernel(x, patch_w, vec32, vec128, wqkv, bqkv, attn_mask, proj_w,
           fc1_w, fc2_w, head_w):
    del attn_mask
    B = x.shape[0]
    L = wqkv.shape[0]

    eye2 = jnp.eye(2, dtype=patch_w.dtype)
    w4 = patch_w.reshape(_CHANS, _PATCH, _PATCH, _D)
    wbig = jnp.einsum('cabe,hH,wV->chawbHVe', w4, eye2, eye2)
    wbig = wbig.reshape(_KFLAT, (_N - 1) * _D)

    wq = wqkv.reshape(L, _D, _H, 3, _HD).transpose(0, 1, 3, 2, 4)
    wq = wq.reshape(L, _D, 3 * _D)
    bq = bqkv.reshape(L, 1, _H, 3, _HD).transpose(0, 1, 3, 2, 4)
    bq = bq.reshape(L, 1, 3 * _D)

    red = jnp.asarray(np.kron(np.eye(_H), np.ones((_HD, _HD))),
                      dtype=jnp.float32)

    x2 = x.reshape(B, _KFLAT)

    tb = 512
    while B % tb:
        tb //= 2
    grid = (B // tb,)

    weights = [wbig, vec32, vec128, wq, bq, red, proj_w, fc1_w, fc2_w, head_w]

    def fixed(a):
        nd = a.ndim
        return pl.BlockSpec(a.shape, lambda i, _nd=nd: (0,) * _nd)

    kern = functools.partial(_vit_kernel, tb=tb, depth=L)
    out = pl.pallas_call(
        kern,
        grid=grid,
        out_shape=jax.ShapeDtypeStruct((B, _HIDDEN), jnp.float32),
        in_specs=[pl.BlockSpec((tb, _KFLAT), lambda i: (i, 0))] +
                 [fixed(a) for a in weights],
        out_specs=pl.BlockSpec((tb, _HIDDEN), lambda i: (i, 0)),
        compiler_params=pltpu.CompilerParams(
            dimension_semantics=("parallel",)),
    )(x2, *weights)
    return out[:, :_NUM_CLASSES]


# speedup vs baseline: 76.6779x; 1.0107x over previous
import functools
import numpy as np
import jax
import jax.numpy as jnp
from jax.experimental import pallas as pl
from jax.experimental.pallas import tpu as pltpu

_D = 32
_H = 4
_HD = _D // _H
_N = 5
_PATCH = 16
_CHANS = 3
_IMG = 32
_KFLAT = _CHANS * _IMG * _IMG
_HIDDEN = 128
_NUM_CLASSES = 10
_EPS = 1e-6
_GELU_C = float(np.sqrt(2.0 / np.pi))


def _layernorm(v, w, b):
    mu = jnp.mean(v, axis=-1, keepdims=True)
    d = v - mu
    var = jnp.mean(d * d, axis=-1, keepdims=True)
    return d * jax.lax.rsqrt(var + _EPS) * w + b


def _gelu_tanh(v):
    return 0.5 * v * (1.0 + jnp.tanh(_GELU_C * (v + 0.044715 * v * v * v)))


def _vit_kernel(xb_ref, wbig_ref, vec32_ref, vec128_ref, wqkv_ref, bqkv_ref,
                red_ref, projw_ref, fc1w_ref, fc2w_ref, headw_ref, o_ref,
                *, tb, depth):
    scale = float(_HD) ** -0.5

    def vrow(r):
        return vec32_ref[pl.ds(r, 1), :]

    emb = jnp.dot(xb_ref[...], wbig_ref[...],
                  preferred_element_type=jnp.float32)

    toks = [jnp.broadcast_to(vrow(0), (tb, _D))]
    for p in range(_N - 1):
        toks.append(emb[:, p * _D:(p + 1) * _D] + vrow(p + 1))
    x = jnp.concatenate(toks, axis=0)

    for l in range(depth):
        base = _N + 6 * l
        h = _layernorm(x, vrow(base), vrow(base + 1))
        qkv = jnp.dot(h, wqkv_ref[l],
                      preferred_element_type=jnp.float32) + bqkv_ref[l]
        q = qkv[:, 0:_D] * scale
        k = qkv[:, _D:2 * _D]
        v = qkv[:, 2 * _D:3 * _D]

        ctxs = []
        for a in range(_N):
            qa = q[a * tb:(a + 1) * tb]
            logits = [jnp.dot(qa * k[b * tb:(b + 1) * tb], red_ref[...],
                              preferred_element_type=jnp.float32)
                      for b in range(_N)]
            m = logits[0]
            for b in range(1, _N):
                m = jnp.maximum(m, logits[b])
            exps = [jnp.exp(lg - m) for lg in logits]
            z = exps[0]
            for b in range(1, _N):
                z = z + exps[b]
            num = exps[0] * v[0:tb]
            for b in range(1, _N):
                num = num + exps[b] * v[b * tb:(b + 1) * tb]
            ctxs.append(num / z)
        ctx = jnp.concatenate(ctxs, axis=0)
        x = x + jnp.dot(ctx, projw_ref[l],
                        preferred_element_type=jnp.float32) + vrow(base + 2)

        hm = _layernorm(x, vrow(base + 3), vrow(base + 4))
        hm = jnp.dot(hm, fc1w_ref[l],
                     preferred_element_type=jnp.float32) + vec128_ref[pl.ds(l, 1), :]
        hm = _gelu_tanh(hm)
        x = x + jnp.dot(hm, fc2w_ref[l],
                        preferred_element_type=jnp.float32) + vrow(base + 5)

    nrow = _N + 6 * depth
    cls = _layernorm(x[0:tb], vrow(nrow), vrow(nrow + 1))
    logits = jnp.dot(cls, headw_ref[...],
                     preferred_element_type=jnp.float32) + vec128_ref[pl.ds(depth, 1), :]
    o_ref[...] = logits[:, :_NUM_CLASSES]


def kernel(x, patch_w, vec32, vec128, wqkv, bqkv, attn_mask, proj_w,
           fc1_w, fc2_w, head_w):
    del attn_mask
    B = x.shape[0]
    L = wqkv.shape[0]

    eye2 = jnp.eye(2, dtype=patch_w.dtype)
    w4 = patch_w.reshape(_CHANS, _PATCH, _PATCH, _D)
    wbig = jnp.einsum('cabe,hH,wV->chawbHVe', w4, eye2, eye2)
    wbig = wbig.reshape(_KFLAT, (_N - 1) * _D).astype(jnp.bfloat16)

    wq = wqkv.reshape(L, _D, _H, 3, _HD).transpose(0, 1, 3, 2, 4)
    wq = wq.reshape(L, _D, 3 * _D)
    bq = bqkv.reshape(L, 1, _H, 3, _HD).transpose(0, 1, 3, 2, 4)
    bq = bq.reshape(L, 1, 3 * _D)

    red = jnp.asarray(np.kron(np.eye(_H), np.ones((_HD, _HD))),
                      dtype=jnp.float32)

    x2 = x.reshape(B, _KFLAT).astype(jnp.bfloat16)

    tb = 512
    while B % tb:
        tb //= 2
    grid = (B // tb,)

    weights = [wbig, vec32, vec128, wq, bq, red, proj_w, fc1_w, fc2_w, head_w]

    def fixed(a):
        nd = a.ndim
        return pl.BlockSpec(a.shape, lambda i, _nd=nd: (0,) * _nd)

    kern = functools.partial(_vit_kernel, tb=tb, depth=L)
    out = pl.pallas_call(
        kern,
        grid=grid,
        out_shape=jax.ShapeDtypeStruct((B, _NUM_CLASSES), jnp.float32),
        in_specs=[pl.BlockSpec((tb, _KFLAT), lambda i: (i, 0))] +
                 [fixed(a) for a in weights],
        out_specs=pl.BlockSpec((tb, _NUM_CLASSES), lambda i: (i, 0)),
        compiler_params=pltpu.CompilerParams(
            dimension_semantics=("parallel",)),
    )(x2, *weights)
    return out
